# baseline (speedup 1.0000x reference)
import functools

import jax
import jax.numpy as jnp
from jax import lax
from jax.experimental import pallas as pl
from jax.experimental.pallas import tpu as pltpu
from jax.experimental.pallas import tpu_sc as plsc

T, D, F, E = 2048, 1024, 512, 8
BM = 128
NB = T // BM
NT = NB + E - 1
NW = 32
CHUNK = T // NW


def _silu(x):
    return x * jax.nn.sigmoid(x)


def _router_body(x_ref, wr_ref, pos_ref, w_ref, counts_ref):
    x = x_ref[...]
    logits = jnp.dot(x, wr_ref[...], preferred_element_type=jnp.float32)
    idx = jnp.argmax(logits, axis=1)
    w_ref[...] = jax.nn.sigmoid(jnp.max(logits, axis=1))
    onehot = (jax.lax.broadcasted_iota(jnp.int32, (E, T), 0)
              == idx[None, :]).astype(jnp.int32)
    csum = onehot
    k = 1
    while k < T:
        shifted = jnp.concatenate(
            [jnp.zeros((E, k), jnp.int32), csum[:, :T - k]], axis=1)
        csum = csum + shifted
        k *= 2
    counts = csum[:, T - 1]
    ir = jax.lax.broadcasted_iota(jnp.int32, (E, E), 0)
    ic = jax.lax.broadcasted_iota(jnp.int32, (E, E), 1)
    off = jnp.sum(jnp.where(ir < ic, counts[:, None], 0), axis=0)
    rank = jnp.sum(jnp.where(onehot == 1, csum - 1, 0), axis=0)
    base = jnp.sum(jnp.where(onehot == 1, off[:, None], 0), axis=0)
    pos_ref[...] = rank + base
    counts_ref[...] = counts


def _router(x, Wr):
    return pl.pallas_call(
        _router_body,
        out_shape=(
            jax.ShapeDtypeStruct((T,), jnp.int32),
            jax.ShapeDtypeStruct((T,), jnp.float32),
            jax.ShapeDtypeStruct((E,), jnp.int32),
        ),
    )(x, Wr)


@functools.cache
def _sc_kernels():
    mesh = plsc.VectorSubcoreMesh(core_axis_name="c", subcore_axis_name="s")
    scratch = [
        pltpu.VMEM((CHUNK,), jnp.int32),
        pltpu.VMEM((CHUNK, D), jnp.float32),
        pltpu.SemaphoreType.DMA,
    ]

    @functools.partial(
        pl.kernel,
        out_type=jax.ShapeDtypeStruct((T, D), jnp.float32),
        mesh=mesh,
        scratch_types=scratch,
    )
    def sc_scatter(x_hbm, pos_hbm, xs_hbm, idx_v, rows_v, sem):
        wid = lax.axis_index("s") * 2 + lax.axis_index("c")
        base = wid * CHUNK
        pltpu.sync_copy(pos_hbm.at[pl.ds(base, CHUNK)], idx_v)
        pltpu.sync_copy(x_hbm.at[pl.ds(base, CHUNK)], rows_v)
        pltpu.async_copy(rows_v, xs_hbm.at[idx_v], sem).wait()

    @functools.partial(
        pl.kernel,
        out_type=jax.ShapeDtypeStruct((T, D), jnp.float32),
        mesh=mesh,
        scratch_types=scratch,
    )
    def sc_gather(ys_hbm, pos_hbm, ysn_hbm, idx_v, rows_v, sem):
        wid = lax.axis_index("s") * 2 + lax.axis_index("c")
        base = wid * CHUNK
        pltpu.sync_copy(pos_hbm.at[pl.ds(base, CHUNK)], idx_v)
        pltpu.async_copy(ys_hbm.at[idx_v], rows_v, sem).wait()
        pltpu.sync_copy(rows_v, ysn_hbm.at[pl.ds(base, CHUNK)])

    return sc_scatter, sc_gather


def _sc_scatter(x, pos):
    return _sc_kernels()[0](x, pos)


def _sc_gather(ys, pos):
    return _sc_kernels()[1](ys, pos)


def _shared_body(x_ref, sg_ref, su_ref, sd_ref, out_ref):
    x = x_ref[...]
    g = jnp.dot(x, sg_ref[...], preferred_element_type=jnp.float32)
    u = jnp.dot(x, su_ref[...], preferred_element_type=jnp.float32)
    out_ref[...] = jnp.dot(_silu(g) * u, sd_ref[...],
                           preferred_element_type=jnp.float32)


def _shared(x, Sg, Su, Sd):
    return pl.pallas_call(
        _shared_body,
        grid=(NB // 2,),
        in_specs=[
            pl.BlockSpec((2 * BM, D), lambda i: (i, 0)),
            pl.BlockSpec((D, F), lambda i: (0, 0)),
            pl.BlockSpec((D, F), lambda i: (0, 0)),
            pl.BlockSpec((F, D), lambda i: (0, 0)),
        ],
        out_specs=pl.BlockSpec((2 * BM, D), lambda i: (i, 0)),
        out_shape=jax.ShapeDtypeStruct((T, D), jnp.float32),
    )(x, Sg, Su, Sd)


def _group_body(m_ref, xs_ref, wg_ref, wu_ref, wd_ref, ys_ref):
    i = pl.program_id(0)
    rb = m_ref[0, i]
    lo = m_ref[2, i]
    hi = m_ref[3, i]
    x = xs_ref[...]
    g = jnp.dot(x, wg_ref[0], preferred_element_type=jnp.float32)
    u = jnp.dot(x, wu_ref[0], preferred_element_type=jnp.float32)
    y = jnp.dot(_silu(g) * u, wd_ref[0], preferred_element_type=jnp.float32)
    row = rb * BM + jax.lax.broadcasted_iota(jnp.int32, (BM, 1), 0)
    contrib = jnp.where((row >= lo) & (row < hi), y, 0.0)
    prev_rb = m_ref[0, jnp.maximum(i - 1, 0)]
    first = (i == 0) | (rb != prev_rb)

    @pl.when(first)
    def _init():
        ys_ref[...] = contrib

    @pl.when(jnp.logical_not(first))
    def _acc():
        ys_ref[...] += contrib


def _grouped(meta, xs, Wg, Wu, Wd):
    grid_spec = pltpu.PrefetchScalarGridSpec(
        num_scalar_prefetch=1,
        grid=(NT,),
        in_specs=[
            pl.BlockSpec((BM, D), lambda i, m: (m[0, i], 0)),
            pl.BlockSpec((1, D, F), lambda i, m: (m[1, i], 0, 0)),
            pl.BlockSpec((1, D, F), lambda i, m: (m[1, i], 0, 0)),
            pl.BlockSpec((1, F, D), lambda i, m: (m[1, i], 0, 0)),
        ],
        out_specs=pl.BlockSpec((BM, D), lambda i, m: (m[0, i], 0)),
    )
    return pl.pallas_call(
        _group_body,
        grid_spec=grid_spec,
        out_shape=jax.ShapeDtypeStruct((T, D), jnp.float32),
    )(meta, xs, Wg, Wu, Wd)


def _build_meta(counts):
    csum = jnp.cumsum(counts)
    off = csum - counts
    b = jnp.arange(NB, dtype=jnp.int32)[:, None]
    lo_e = off[None, :].astype(jnp.int32)
    hi_e = csum[None, :].astype(jnp.int32)
    active = ((hi_e > b * BM) & (lo_e < (b + 1) * BM)
              & (counts[None, :] > 0))
    flat = jnp.nonzero(active.ravel(), size=NT, fill_value=NB * E - 1)[0]
    nact = jnp.sum(active.astype(jnp.int32))
    pad = jnp.arange(NT, dtype=jnp.int32) >= nact
    rb = jnp.where(pad, NB - 1, flat // E).astype(jnp.int32)
    ex = jnp.where(pad, E - 1, flat % E).astype(jnp.int32)
    lo = jnp.where(pad, 0, off[ex]).astype(jnp.int32)
    hi = jnp.where(pad, 0, csum[ex]).astype(jnp.int32)
    return jnp.stack([rb, ex, lo, hi])


def _combine_body(sh_ref, ysn_ref, w_ref, out_ref):
    out_ref[...] = sh_ref[...] + w_ref[...][:, None] * ysn_ref[...]


def _combine(shared, ysn, w):
    return pl.pallas_call(
        _combine_body,
        grid=(NB // 2,),
        in_specs=[
            pl.BlockSpec((2 * BM, D), lambda i: (i, 0)),
            pl.BlockSpec((2 * BM, D), lambda i: (i, 0)),
            pl.BlockSpec((2 * BM,), lambda i: (i,)),
        ],
        out_specs=pl.BlockSpec((2 * BM, D), lambda i: (i, 0)),
        out_shape=jax.ShapeDtypeStruct((T, D), jnp.float32),
    )(shared, ysn, w)


@jax.jit
def kernel(hidden_states, Wr, Wg, Wu, Wd, Sg, Su, Sd):
    pos, w, counts = _router(hidden_states, Wr)
    meta = _build_meta(counts)
    xs = _sc_scatter(hidden_states, pos)
    shared = _shared(hidden_states, Sg, Su, Sd)
    ys = _grouped(meta, xs, Wg, Wu, Wd)
    ysn = _sc_gather(ys, pos)
    return _combine(shared, ysn, w)


# speedup vs baseline: 1.0615x; 1.0615x over previous
import functools

import jax
import jax.numpy as jnp
from jax import lax
from jax.experimental import pallas as pl
from jax.experimental.pallas import tpu as pltpu
from jax.experimental.pallas import tpu_sc as plsc

T, D, F, E = 2048, 1024, 512, 8
BM = 128
NB = T // BM
NT = NB + E - 1
NW = 32
CHUNK = T // NW
LANES = 16


def _silu(x):
    return x * jax.nn.sigmoid(x)


def _router_body(x_ref, wr_ref, pos_ref, meta_ref):
    x = x_ref[...]
    logits = jnp.dot(x, wr_ref[...], preferred_element_type=jnp.float32)
    idx = jnp.argmax(logits, axis=1)
    onehot = (jax.lax.broadcasted_iota(jnp.int32, (E, T), 0)
              == idx[None, :]).astype(jnp.int32)
    csum = onehot
    k = 1
    while k < T:
        csum = csum + jnp.concatenate(
            [jnp.zeros((E, k), jnp.int32), csum[:, :T - k]], axis=1)
        k *= 2
    counts = csum[:, T - 1]
    ir = jax.lax.broadcasted_iota(jnp.int32, (E, E), 0)
    ic = jax.lax.broadcasted_iota(jnp.int32, (E, E), 1)
    off = jnp.sum(jnp.where(ir < ic, counts[:, None], 0), axis=0)
    seg_hi = off + counts
    rank = jnp.sum(jnp.where(onehot == 1, csum - 1, 0), axis=0)
    base = jnp.sum(jnp.where(onehot == 1, off[:, None], 0), axis=0)
    pos_ref[...] = rank + base

    bcol = jax.lax.broadcasted_iota(jnp.int32, (NB, E), 0) * BM
    act = ((seg_hi[None, :] > bcol) & (off[None, :] < bcol + BM)
           & (counts[None, :] > 0)).astype(jnp.int32)
    srow = act
    k = 1
    while k < E:
        srow = srow + jnp.concatenate(
            [jnp.zeros((NB, k), jnp.int32), srow[:, :E - k]], axis=1)
        k *= 2
    rowtot = srow[:, E - 1:E]
    rcs = rowtot
    k = 1
    while k < NB:
        rcs = rcs + jnp.concatenate(
            [jnp.zeros((k, 1), jnp.int32), rcs[:NB - k, :]], axis=0)
        k *= 2
    s_flat = srow + (rcs - rowtot)
    nact = rcs[NB - 1, 0]

    jj = jax.lax.broadcasted_iota(jnp.int32, (NT, NB, E), 0)
    m = ((act[None] == 1) & (s_flat[None] == jj + 1)).astype(jnp.int32)
    b3 = jax.lax.broadcasted_iota(jnp.int32, (NT, NB, E), 1)
    e3 = jax.lax.broadcasted_iota(jnp.int32, (NT, NB, E), 2)
    rb = jnp.sum(m * b3, axis=(1, 2))
    ex = jnp.sum(m * e3, axis=(1, 2))
    lo = jnp.sum(m * jnp.broadcast_to(off[None, None, :], (NT, NB, E)),
                 axis=(1, 2))
    hi = jnp.sum(m * jnp.broadcast_to(seg_hi[None, None, :], (NT, NB, E)),
                 axis=(1, 2))
    pad = jax.lax.broadcasted_iota(jnp.int32, (NT,), 0) >= nact
    rb = jnp.where(pad, NB - 1, rb)
    ex = jnp.where(pad, E - 1, ex)
    lo = jnp.where(pad, 0, lo)
    hi = jnp.where(pad, 0, hi)
    meta_ref[...] = jnp.concatenate(
        [rb[None, :], ex[None, :], lo[None, :], hi[None, :]], axis=0)


def _router(x, Wr):
    return pl.pallas_call(
        _router_body,
        out_shape=(
            jax.ShapeDtypeStruct((T,), jnp.int32),
            jax.ShapeDtypeStruct((4, NT), jnp.int32),
        ),
    )(x, Wr)


@functools.cache
def _sc_kernels():
    mesh = plsc.VectorSubcoreMesh(core_axis_name="c", subcore_axis_name="s")

    @functools.partial(
        pl.kernel,
        out_type=jax.ShapeDtypeStruct((T, D), jnp.float32),
        mesh=mesh,
        scratch_types=[
            pltpu.VMEM((CHUNK,), jnp.int32),
            pltpu.VMEM((CHUNK, D), jnp.float32),
            pltpu.SemaphoreType.DMA,
        ],
    )
    def sc_scatter(x_hbm, pos_hbm, xs_hbm, idx_v, rows_v, sem):
        wid = lax.axis_index("s") * 2 + lax.axis_index("c")
        base = wid * CHUNK
        pltpu.sync_copy(pos_hbm.at[pl.ds(base, CHUNK)], idx_v)
        pltpu.sync_copy(x_hbm.at[pl.ds(base, CHUNK)], rows_v)
        pltpu.async_copy(rows_v, xs_hbm.at[idx_v], sem).wait()

    @functools.partial(
        pl.kernel,
        out_type=jax.ShapeDtypeStruct((T, D), jnp.float32),
        mesh=mesh,
        scratch_types=[
            pltpu.VMEM((CHUNK,), jnp.int32),
            pltpu.VMEM((CHUNK, D), jnp.float32),
            pltpu.VMEM((CHUNK // 2, D), jnp.float32),
            pltpu.SemaphoreType.DMA,
        ],
    )
    def sc_gather_add(ys_hbm, pos_hbm, sh_hbm, out_hbm, idx_v, rows_v, sh_v,
                      sem):
        wid = lax.axis_index("s") * 2 + lax.axis_index("c")
        base = wid * CHUNK
        half = CHUNK // 2
        pltpu.sync_copy(pos_hbm.at[pl.ds(base, CHUNK)], idx_v)
        pltpu.async_copy(ys_hbm.at[idx_v], rows_v, sem).wait()
        for h in range(2):
            pltpu.sync_copy(sh_hbm.at[pl.ds(base + h * half, half)], sh_v)

            def row_add(r, _):
                for c in range(D // LANES):
                    sl = pl.ds(c * LANES, LANES)
                    rr = h * half + r
                    rows_v[rr, sl] = rows_v[rr, sl] + sh_v[r, sl]
                return 0

            lax.fori_loop(0, half, row_add, 0)
        pltpu.sync_copy(rows_v, out_hbm.at[pl.ds(base, CHUNK)])

    return sc_scatter, sc_gather_add


def _sc_scatter(x, pos):
    return _sc_kernels()[0](x, pos)


def _sc_gather_add(ys, pos, shared):
    return _sc_kernels()[1](ys, pos, shared)


def _shared_body(x_ref, sg_ref, su_ref, sd_ref, out_ref):
    x = x_ref[...]
    g = jnp.dot(x, sg_ref[...], preferred_element_type=jnp.float32)
    u = jnp.dot(x, su_ref[...], preferred_element_type=jnp.float32)
    out_ref[...] = jnp.dot(_silu(g) * u, sd_ref[...],
                           preferred_element_type=jnp.float32)


def _shared(x, Sg, Su, Sd):
    return pl.pallas_call(
        _shared_body,
        grid=(NB // 2,),
        in_specs=[
            pl.BlockSpec((2 * BM, D), lambda i: (i, 0)),
            pl.BlockSpec((D, F), lambda i: (0, 0)),
            pl.BlockSpec((D, F), lambda i: (0, 0)),
            pl.BlockSpec((F, D), lambda i: (0, 0)),
        ],
        out_specs=pl.BlockSpec((2 * BM, D), lambda i: (i, 0)),
        out_shape=jax.ShapeDtypeStruct((T, D), jnp.float32),
    )(x, Sg, Su, Sd)


def _group_body(m_ref, xs_ref, wr_ref, wg_ref, wu_ref, wd_ref, ys_ref):
    i = pl.program_id(0)
    rb = m_ref[0, i]
    lo = m_ref[2, i]
    hi = m_ref[3, i]
    x = xs_ref[...]
    logits = jnp.dot(x, wr_ref[...], preferred_element_type=jnp.float32)
    ws = jax.nn.sigmoid(jnp.max(logits, axis=1))
    g = jnp.dot(x, wg_ref[0], preferred_element_type=jnp.float32)
    u = jnp.dot(x, wu_ref[0], preferred_element_type=jnp.float32)
    y = jnp.dot(_silu(g) * u, wd_ref[0], preferred_element_type=jnp.float32)
    row = rb * BM + jax.lax.broadcasted_iota(jnp.int32, (BM, 1), 0)
    contrib = jnp.where((row >= lo) & (row < hi), ws[:, None] * y, 0.0)
    prev_rb = m_ref[0, jnp.maximum(i - 1, 0)]
    first = (i == 0) | (rb != prev_rb)

    @pl.when(first)
    def _init():
        ys_ref[...] = contrib

    @pl.when(jnp.logical_not(first))
    def _acc():
        ys_ref[...] += contrib


def _grouped(meta, xs, Wr, Wg, Wu, Wd):
    grid_spec = pltpu.PrefetchScalarGridSpec(
        num_scalar_prefetch=1,
        grid=(NT,),
        in_specs=[
            pl.BlockSpec((BM, D), lambda i, m: (m[0, i], 0)),
            pl.BlockSpec((D, E), lambda i, m: (0, 0)),
            pl.BlockSpec((1, D, F), lambda i, m: (m[1, i], 0, 0)),
            pl.BlockSpec((1, D, F), lambda i, m: (m[1, i], 0, 0)),
            pl.BlockSpec((1, F, D), lambda i, m: (m[1, i], 0, 0)),
        ],
        out_specs=pl.BlockSpec((BM, D), lambda i, m: (m[0, i], 0)),
    )
    return pl.pallas_call(
        _group_body,
        grid_spec=grid_spec,
        out_shape=jax.ShapeDtypeStruct((T, D), jnp.float32),
    )(meta, xs, Wr, Wg, Wu, Wd)


@jax.jit
def kernel(hidden_states, Wr, Wg, Wu, Wd, Sg, Su, Sd):
    pos, meta = _router(hidden_states, Wr)
    xs = _sc_scatter(hidden_states, pos)
    shared = _shared(hidden_states, Sg, Su, Sd)
    ys = _grouped(meta, xs, Wr, Wg, Wu, Wd)
    return _sc_gather_add(ys, pos, shared)


# speedup vs baseline: 1.2223x; 1.1515x over previous
import functools

import jax
import jax.numpy as jnp
from jax import lax
from jax.experimental import pallas as pl
from jax.experimental.pallas import tpu as pltpu
from jax.experimental.pallas import tpu_sc as plsc

T, D, F, E = 2048, 1024, 512, 8
BM = 256
NB = T // BM
NT = NB + E - 1
NW = 32
CHUNK = T // NW


def _silu(x):
    return x * jax.nn.sigmoid(x)


def _router_body(x_ref, wr_ref, pos_ref, meta_ref):
    x = x_ref[...]
    logits = jnp.dot(x, wr_ref[...], preferred_element_type=jnp.float32)
    idx = jnp.argmax(logits, axis=1)
    onehot = (jax.lax.broadcasted_iota(jnp.int32, (E, T), 0)
              == idx[None, :]).astype(jnp.int32)
    csum = onehot
    k = 1
    while k < T:
        csum = csum + jnp.concatenate(
            [jnp.zeros((E, k), jnp.int32), csum[:, :T - k]], axis=1)
        k *= 2
    counts = csum[:, T - 1]
    ir = jax.lax.broadcasted_iota(jnp.int32, (E, E), 0)
    ic = jax.lax.broadcasted_iota(jnp.int32, (E, E), 1)
    off = jnp.sum(jnp.where(ir < ic, counts[:, None], 0), axis=0)
    seg_hi = off + counts
    rank = jnp.sum(jnp.where(onehot == 1, csum - 1, 0), axis=0)
    base = jnp.sum(jnp.where(onehot == 1, off[:, None], 0), axis=0)
    pos_ref[...] = rank + base

    bcol = jax.lax.broadcasted_iota(jnp.int32, (NB, E), 0) * BM
    act = ((seg_hi[None, :] > bcol) & (off[None, :] < bcol + BM)
           & (counts[None, :] > 0)).astype(jnp.int32)
    srow = act
    k = 1
    while k < E:
        srow = srow + jnp.concatenate(
            [jnp.zeros((NB, k), jnp.int32), srow[:, :E - k]], axis=1)
        k *= 2
    rowtot = srow[:, E - 1:E]
    rcs = rowtot
    k = 1
    while k < NB:
        rcs = rcs + jnp.concatenate(
            [jnp.zeros((k, 1), jnp.int32), rcs[:NB - k, :]], axis=0)
        k *= 2
    s_flat = srow + (rcs - rowtot)
    nact = rcs[NB - 1, 0]

    jj = jax.lax.broadcasted_iota(jnp.int32, (NT, NB, E), 0)
    m = ((act[None] == 1) & (s_flat[None] == jj + 1)).astype(jnp.int32)
    b3 = jax.lax.broadcasted_iota(jnp.int32, (NT, NB, E), 1)
    e3 = jax.lax.broadcasted_iota(jnp.int32, (NT, NB, E), 2)
    rb = jnp.sum(m * b3, axis=(1, 2))
    ex = jnp.sum(m * e3, axis=(1, 2))
    lo = jnp.sum(m * jnp.broadcast_to(off[None, None, :], (NT, NB, E)),
                 axis=(1, 2))
    hi = jnp.sum(m * jnp.broadcast_to(seg_hi[None, None, :], (NT, NB, E)),
                 axis=(1, 2))
    pad = jax.lax.broadcasted_iota(jnp.int32, (NT,), 0) >= nact
    rb = jnp.where(pad, NB - 1, rb)
    ex = jnp.where(pad, E - 1, ex)
    lo = jnp.where(pad, 0, lo)
    hi = jnp.where(pad, 0, hi)
    meta_ref[...] = jnp.concatenate(
        [rb[None, :], ex[None, :], lo[None, :], hi[None, :]], axis=0)


def _router(x, Wr):
    return pl.pallas_call(
        _router_body,
        out_shape=(
            jax.ShapeDtypeStruct((T,), jnp.int32),
            jax.ShapeDtypeStruct((4, NT), jnp.int32),
        ),
    )(x, Wr)


@functools.cache
def _sc_kernels():
    mesh = plsc.VectorSubcoreMesh(core_axis_name="c", subcore_axis_name="s")
    scratch = [
        pltpu.VMEM((CHUNK,), jnp.int32),
        pltpu.VMEM((CHUNK, D), jnp.float32),
        pltpu.SemaphoreType.DMA,
    ]

    @functools.partial(
        pl.kernel,
        out_type=jax.ShapeDtypeStruct((T, D), jnp.float32),
        mesh=mesh,
        scratch_types=scratch,
    )
    def sc_scatter(x_hbm, pos_hbm, xs_hbm, idx_v, rows_v, sem):
        wid = lax.axis_index("s") * 2 + lax.axis_index("c")
        base = wid * CHUNK
        pltpu.sync_copy(pos_hbm.at[pl.ds(base, CHUNK)], idx_v)
        pltpu.sync_copy(x_hbm.at[pl.ds(base, CHUNK)], rows_v)
        pltpu.async_copy(rows_v, xs_hbm.at[idx_v], sem).wait()

    @functools.partial(
        pl.kernel,
        out_type=jax.ShapeDtypeStruct((T, D), jnp.float32),
        mesh=mesh,
        scratch_types=scratch,
    )
    def sc_gather(ys_hbm, pos_hbm, out_hbm, idx_v, rows_v, sem):
        wid = lax.axis_index("s") * 2 + lax.axis_index("c")
        base = wid * CHUNK
        pltpu.sync_copy(pos_hbm.at[pl.ds(base, CHUNK)], idx_v)
        pltpu.async_copy(ys_hbm.at[idx_v], rows_v, sem).wait()
        pltpu.sync_copy(rows_v, out_hbm.at[pl.ds(base, CHUNK)])

    return sc_scatter, sc_gather


def _sc_scatter(x, pos):
    return _sc_kernels()[0](x, pos)


def _sc_gather(ys, pos):
    return _sc_kernels()[1](ys, pos)


def _group_body(m_ref, xs_ref, wr_ref, wg_ref, wu_ref, wd_ref,
                sg_ref, su_ref, sd_ref, ys_ref):
    i = pl.program_id(0)
    rb = m_ref[0, i]
    lo = m_ref[2, i]
    hi = m_ref[3, i]
    x = xs_ref[...]
    logits = jnp.dot(x, wr_ref[...], preferred_element_type=jnp.float32)
    ws = jax.nn.sigmoid(jnp.max(logits, axis=1))
    g = jnp.dot(x, wg_ref[0], preferred_element_type=jnp.float32)
    u = jnp.dot(x, wu_ref[0], preferred_element_type=jnp.float32)
    y = jnp.dot(_silu(g) * u, wd_ref[0], preferred_element_type=jnp.float32)
    sg = jnp.dot(x, sg_ref[...], preferred_element_type=jnp.float32)
    su = jnp.dot(x, su_ref[...], preferred_element_type=jnp.float32)
    sh = jnp.dot(_silu(sg) * su, sd_ref[...],
                 preferred_element_type=jnp.float32)
    row = rb * BM + jax.lax.broadcasted_iota(jnp.int32, (BM, 1), 0)
    contrib = jnp.where((row >= lo) & (row < hi), ws[:, None] * y + sh, 0.0)
    prev_rb = m_ref[0, jnp.maximum(i - 1, 0)]
    first = (i == 0) | (rb != prev_rb)

    @pl.when(first)
    def _init():
        ys_ref[...] = contrib

    @pl.when(jnp.logical_not(first))
    def _acc():
        ys_ref[...] += contrib


def _grouped(meta, xs, Wr, Wg, Wu, Wd, Sg, Su, Sd):
    grid_spec = pltpu.PrefetchScalarGridSpec(
        num_scalar_prefetch=1,
        grid=(NT,),
        in_specs=[
            pl.BlockSpec((BM, D), lambda i, m: (m[0, i], 0)),
            pl.BlockSpec((D, E), lambda i, m: (0, 0)),
            pl.BlockSpec((1, D, F), lambda i, m: (m[1, i], 0, 0)),
            pl.BlockSpec((1, D, F), lambda i, m: (m[1, i], 0, 0)),
            pl.BlockSpec((1, F, D), lambda i, m: (m[1, i], 0, 0)),
            pl.BlockSpec((D, F), lambda i, m: (0, 0)),
            pl.BlockSpec((D, F), lambda i, m: (0, 0)),
            pl.BlockSpec((F, D), lambda i, m: (0, 0)),
        ],
        out_specs=pl.BlockSpec((BM, D), lambda i, m: (m[0, i], 0)),
    )
    return pl.pallas_call(
        _group_body,
        grid_spec=grid_spec,
        out_shape=jax.ShapeDtypeStruct((T, D), jnp.float32),
    )(meta, xs, Wr, Wg, Wu, Wd, Sg, Su, Sd)


@jax.jit
def kernel(hidden_states, Wr, Wg, Wu, Wd, Sg, Su, Sd):
    pos, meta = _router(hidden_states, Wr)
    xs = _sc_scatter(hidden_states, pos)
    ys = _grouped(meta, xs, Wr, Wg, Wu, Wd, Sg, Su, Sd)
    return _sc_gather(ys, pos)


# speedup vs baseline: 1.2225x; 1.0001x over previous
import functools

import jax
import jax.numpy as jnp
from jax import lax
from jax.experimental import pallas as pl
from jax.experimental.pallas import tpu as pltpu
from jax.experimental.pallas import tpu_sc as plsc

T, D, F, E = 2048, 1024, 512, 8
BM = 256
NB = T // BM
NT = NB + E - 1
NW = 32
CHUNK = T // NW


def _silu(x):
    return x * jax.nn.sigmoid(x)


def _router_body(x_ref, wr_ref, pos_ref, meta_ref):
    x = x_ref[...]
    logits = jnp.dot(x, wr_ref[...], preferred_element_type=jnp.float32)
    idx = jnp.argmax(logits, axis=1)
    onehot = (jax.lax.broadcasted_iota(jnp.int32, (E, T), 0)
              == idx[None, :]).astype(jnp.int32)
    csum = onehot
    k = 1
    while k < T:
        csum = csum + jnp.concatenate(
            [jnp.zeros((E, k), jnp.int32), csum[:, :T - k]], axis=1)
        k *= 2
    counts = csum[:, T - 1]
    ir = jax.lax.broadcasted_iota(jnp.int32, (E, E), 0)
    ic = jax.lax.broadcasted_iota(jnp.int32, (E, E), 1)
    off = jnp.sum(jnp.where(ir < ic, counts[:, None], 0), axis=0)
    seg_hi = off + counts
    rank = jnp.sum(jnp.where(onehot == 1, csum - 1, 0), axis=0)
    base = jnp.sum(jnp.where(onehot == 1, off[:, None], 0), axis=0)
    pos_ref[...] = rank + base

    bcol = jax.lax.broadcasted_iota(jnp.int32, (NB, E), 0) * BM
    act = ((seg_hi[None, :] > bcol) & (off[None, :] < bcol + BM)
           & (counts[None, :] > 0)).astype(jnp.int32)
    srow = act
    k = 1
    while k < E:
        srow = srow + jnp.concatenate(
            [jnp.zeros((NB, k), jnp.int32), srow[:, :E - k]], axis=1)
        k *= 2
    rowtot = srow[:, E - 1:E]
    rcs = rowtot
    k = 1
    while k < NB:
        rcs = rcs + jnp.concatenate(
            [jnp.zeros((k, 1), jnp.int32), rcs[:NB - k, :]], axis=0)
        k *= 2
    s_flat = srow + (rcs - rowtot)
    nact = rcs[NB - 1, 0]

    jj = jax.lax.broadcasted_iota(jnp.int32, (NT, NB, E), 0)
    m = ((act[None] == 1) & (s_flat[None] == jj + 1)).astype(jnp.int32)
    b3 = jax.lax.broadcasted_iota(jnp.int32, (NT, NB, E), 1)
    e3 = jax.lax.broadcasted_iota(jnp.int32, (NT, NB, E), 2)
    rb = jnp.sum(m * b3, axis=(1, 2))
    ex = jnp.sum(m * e3, axis=(1, 2))
    lo = jnp.sum(m * jnp.broadcast_to(off[None, None, :], (NT, NB, E)),
                 axis=(1, 2))
    hi = jnp.sum(m * jnp.broadcast_to(seg_hi[None, None, :], (NT, NB, E)),
                 axis=(1, 2))
    pad = jax.lax.broadcasted_iota(jnp.int32, (NT,), 0) >= nact
    rb = jnp.where(pad, NB - 1, rb)
    ex = jnp.where(pad, E - 1, ex)
    lo = jnp.where(pad, 0, lo)
    hi = jnp.where(pad, 0, hi)
    meta_ref[...] = jnp.concatenate(
        [rb[None, :], ex[None, :], lo[None, :], hi[None, :]], axis=0)


def _router(x, Wr):
    return pl.pallas_call(
        _router_body,
        out_shape=(
            jax.ShapeDtypeStruct((T,), jnp.int32),
            jax.ShapeDtypeStruct((4, NT), jnp.int32),
        ),
    )(x, Wr)


@functools.cache
def _sc_kernels():
    mesh = plsc.VectorSubcoreMesh(core_axis_name="c", subcore_axis_name="s")
    scratch = [
        pltpu.VMEM((CHUNK,), jnp.int32),
        pltpu.VMEM((CHUNK, D), jnp.float32),
        pltpu.SemaphoreType.DMA,
    ]

    @functools.partial(
        pl.kernel,
        out_type=jax.ShapeDtypeStruct((T, D), jnp.float32),
        mesh=mesh,
        scratch_types=scratch,
    )
    def sc_scatter(x_hbm, pos_hbm, xs_hbm, idx_v, rows_v, sem):
        wid = lax.axis_index("s") * 2 + lax.axis_index("c")
        base = wid * CHUNK
        pltpu.sync_copy(pos_hbm.at[pl.ds(base, CHUNK)], idx_v)
        pltpu.sync_copy(x_hbm.at[pl.ds(base, CHUNK)], rows_v)
        pltpu.async_copy(rows_v, xs_hbm.at[idx_v], sem).wait()

    @functools.partial(
        pl.kernel,
        out_type=jax.ShapeDtypeStruct((T, D), jnp.float32),
        mesh=mesh,
        scratch_types=scratch,
    )
    def sc_gather(ys_hbm, pos_hbm, out_hbm, idx_v, rows_v, sem):
        wid = lax.axis_index("s") * 2 + lax.axis_index("c")
        base = wid * CHUNK
        pltpu.sync_copy(pos_hbm.at[pl.ds(base, CHUNK)], idx_v)
        pltpu.async_copy(ys_hbm.at[idx_v], rows_v, sem).wait()
        pltpu.sync_copy(rows_v, out_hbm.at[pl.ds(base, CHUNK)])

    return sc_scatter, sc_gather


def _sc_scatter(x, pos):
    return _sc_kernels()[0](x, pos)


def _sc_gather(ys, pos):
    return _sc_kernels()[1](ys, pos)


def _group_body(m_ref, xs_ref, wr_ref, wg_ref, wu_ref, wd_ref,
                sg_ref, su_ref, sd_ref, ys_ref):
    i = pl.program_id(0)
    rb = m_ref[0, i]
    lo = m_ref[2, i]
    hi = m_ref[3, i]
    x = xs_ref[...]
    logits = jnp.dot(x, wr_ref[...], preferred_element_type=jnp.float32)
    ws = jax.nn.sigmoid(jnp.max(logits, axis=1))
    xb = x.astype(jnp.bfloat16)
    g = jnp.dot(xb, wg_ref[0].astype(jnp.bfloat16),
                preferred_element_type=jnp.float32)
    u = jnp.dot(xb, wu_ref[0].astype(jnp.bfloat16),
                preferred_element_type=jnp.float32)
    y = jnp.dot((_silu(g) * u).astype(jnp.bfloat16),
                wd_ref[0].astype(jnp.bfloat16),
                preferred_element_type=jnp.float32)
    sg = jnp.dot(xb, sg_ref[...].astype(jnp.bfloat16),
                 preferred_element_type=jnp.float32)
    su = jnp.dot(xb, su_ref[...].astype(jnp.bfloat16),
                 preferred_element_type=jnp.float32)
    sh = jnp.dot((_silu(sg) * su).astype(jnp.bfloat16),
                 sd_ref[...].astype(jnp.bfloat16),
                 preferred_element_type=jnp.float32)
    row = rb * BM + jax.lax.broadcasted_iota(jnp.int32, (BM, 1), 0)
    contrib = jnp.where((row >= lo) & (row < hi), ws[:, None] * y + sh, 0.0)
    prev_rb = m_ref[0, jnp.maximum(i - 1, 0)]
    first = (i == 0) | (rb != prev_rb)

    @pl.when(first)
    def _init():
        ys_ref[...] = contrib

    @pl.when(jnp.logical_not(first))
    def _acc():
        ys_ref[...] += contrib


def _grouped(meta, xs, Wr, Wg, Wu, Wd, Sg, Su, Sd):
    grid_spec = pltpu.PrefetchScalarGridSpec(
        num_scalar_prefetch=1,
        grid=(NT,),
        in_specs=[
            pl.BlockSpec((BM, D), lambda i, m: (m[0, i], 0)),
            pl.BlockSpec((D, E), lambda i, m: (0, 0)),
            pl.BlockSpec((1, D, F), lambda i, m: (m[1, i], 0, 0)),
            pl.BlockSpec((1, D, F), lambda i, m: (m[1, i], 0, 0)),
            pl.BlockSpec((1, F, D), lambda i, m: (m[1, i], 0, 0)),
            pl.BlockSpec((D, F), lambda i, m: (0, 0)),
            pl.BlockSpec((D, F), lambda i, m: (0, 0)),
            pl.BlockSpec((F, D), lambda i, m: (0, 0)),
        ],
        out_specs=pl.BlockSpec((BM, D), lambda i, m: (m[0, i], 0)),
    )
    return pl.pallas_call(
        _group_body,
        grid_spec=grid_spec,
        out_shape=jax.ShapeDtypeStruct((T, D), jnp.float32),
    )(meta, xs, Wr, Wg, Wu, Wd, Sg, Su, Sd)


@jax.jit
def kernel(hidden_states, Wr, Wg, Wu, Wd, Sg, Su, Sd):
    pos, meta = _router(hidden_states, Wr)
    xs = _sc_scatter(hidden_states, pos)
    ys = _grouped(meta, xs, Wr, Wg, Wu, Wd, Sg, Su, Sd)
    return _sc_gather(ys, pos)
